# baseline (speedup 1.0000x reference)
import functools

import jax
import jax.numpy as jnp
from jax import lax
from jax.experimental import pallas as pl
from jax.experimental.pallas import tpu as pltpu
from jax.experimental.pallas import tpu_sc as plsc

NC = 2
NS = 16
NW = NC * NS
CHUNK = 128
NBUF = 6
PREFETCH = 3
HIST_BR = 256


def _sc_partial_sums(feats, labels1d, zeros_acc, c):
    n, d = feats.shape
    nblk = n // CHUNK
    blk_per_tile = nblk // NW
    extra = nblk - blk_per_tile * NW
    mesh = plsc.VectorSubcoreMesh(
        core_axis_name="c", subcore_axis_name="s", num_cores=NC, num_subcores=NS)

    @functools.partial(
        pl.kernel,
        out_type=jax.ShapeDtypeStruct((NC, NS * c, d), jnp.float32),
        mesh=mesh,
        scratch_types=[
            pltpu.VMEM((NBUF, CHUNK), jnp.int32),
            pltpu.VMEM((NBUF, CHUNK, d), jnp.float32),
            pltpu.VMEM_SHARED((NS * c, d), jnp.float32),
            pltpu.SemaphoreType.DMA((NBUF,)),
            pltpu.SemaphoreType.DMA((NBUF,)),
            pltpu.SemaphoreType.DMA((NBUF,)),
        ],
    )
    def run(feats_hbm, labels_hbm, zacc_hbm, out_sums, idxb, bufb,
            shared_acc, lsem, fsem, ssem):
        cid = lax.axis_index("c")
        sid = lax.axis_index("s")
        wid = cid * NS + sid
        @pl.when(sid == 0)
        def _():
            pltpu.sync_copy(zacc_hbm, shared_acc)

        plsc.subcore_barrier()

        def start(j):
            p = lax.rem(j, NBUF)
            blk = wid * blk_per_tile + j
            pltpu.async_copy(
                labels_hbm.at[pl.ds(blk * CHUNK, CHUNK)], idxb.at[p],
                lsem.at[p])
            pltpu.async_copy(
                feats_hbm.at[pl.ds(blk * CHUNK, CHUNK)], bufb.at[p],
                fsem.at[p])

        def wait_scatter(p):
            pltpu.make_async_copy(
                bufb.at[p], shared_acc.at[idxb.at[p]], ssem.at[p]).wait()

        for j0 in range(PREFETCH):
            start(j0)

        def body(j, carry):
            p = lax.rem(j, NBUF)
            pltpu.make_async_copy(
                labels_hbm.at[pl.ds(0, CHUNK)], idxb.at[p], lsem.at[p]).wait()
            pltpu.make_async_copy(
                feats_hbm.at[pl.ds(0, CHUNK)], bufb.at[p], fsem.at[p]).wait()
            pltpu.async_copy(bufb.at[p], shared_acc.at[idxb.at[p]], ssem.at[p],
                             add=True)
            m = j + PREFETCH

            @pl.when(m < blk_per_tile)
            def _():
                q = lax.rem(m, NBUF)

                @pl.when(m >= NBUF)
                def _():
                    wait_scatter(q)

                start(m)

            return carry

        lax.fori_loop(0, blk_per_tile, body, 0)
        for t in range(NBUF):
            wait_scatter((blk_per_tile - NBUF + t) % NBUF)

        @pl.when(wid < extra)
        def _():
            blk = blk_per_tile * NW + wid
            pltpu.sync_copy(labels_hbm.at[pl.ds(blk * CHUNK, CHUNK)],
                            idxb.at[0])
            pltpu.sync_copy(feats_hbm.at[pl.ds(blk * CHUNK, CHUNK)],
                            bufb.at[0])
            pltpu.sync_copy(bufb.at[0], shared_acc.at[idxb.at[0]], add=True)

        plsc.subcore_barrier()

        @pl.when(sid == 0)
        def _():
            pltpu.sync_copy(shared_acc, out_sums.at[cid])

    return run(feats, labels1d, zeros_acc)


def _hist_kernel(lab_ref, out_ref):
    i = pl.program_id(0)
    lab = lab_ref[...]
    class_row = lax.broadcasted_iota(jnp.int32, (HIST_BR, 128), 1)
    acc = jnp.zeros((1, 128), jnp.float32)
    for k in range(128):
        col = jnp.broadcast_to(lab[:, k:k + 1], (HIST_BR, 128))
        oh = (col == class_row).astype(jnp.float32)
        acc = acc + jnp.sum(oh, axis=0, keepdims=True)

    @pl.when(i == 0)
    def _():
        out_ref[...] = jnp.zeros_like(out_ref)

    out_ref[...] += jnp.broadcast_to(acc, out_ref.shape)


def _tc_histogram(labels_pad2d):
    rows = labels_pad2d.shape[0]
    return pl.pallas_call(
        _hist_kernel,
        grid=(rows // HIST_BR,),
        in_specs=[pl.BlockSpec((HIST_BR, 128), lambda i: (i, 0))],
        out_specs=pl.BlockSpec((8, 128), lambda i: (0, 0)),
        out_shape=jax.ShapeDtypeStruct((8, 128), jnp.float32),
    )(labels_pad2d)


def _combine_kernel(psums_ref, hist_ref, proto_ref, plabel_ref, out_ref):
    sums = jnp.sum(psums_ref[...], axis=0)
    c = sums.shape[0]
    d = sums.shape[1]
    labels = plabel_ref[...]
    onehot_c = (labels[:, None] ==
                lax.broadcasted_iota(jnp.int32, (c, c), 1)).astype(jnp.float32)
    onehot_k = (labels[:, None] ==
                lax.broadcasted_iota(jnp.int32, (c, 128), 1)).astype(jnp.float32)
    hist_sq = jnp.broadcast_to(hist_ref[0:1, :], (128, 128))
    eye = (lax.broadcasted_iota(jnp.int32, (128, 128), 0) ==
           lax.broadcasted_iota(jnp.int32, (128, 128), 1))
    hist_col = jnp.sum(jnp.where(eye, hist_sq, 0.0), axis=1,
                       keepdims=True)
    sums_sel = jnp.dot(onehot_c, sums,
                       preferred_element_type=jnp.float32,
                       precision=lax.Precision.HIGHEST)
    cnts_sel = jnp.dot(onehot_k, hist_col,
                       preferred_element_type=jnp.float32,
                       precision=lax.Precision.HIGHEST)
    safe = jnp.maximum(cnts_sel, 1.0)
    proto_cur = sums_sel / safe
    present = (cnts_sel > 0).astype(jnp.float32)
    sq = (proto_ref[...] - proto_cur) ** 2
    per_class = jnp.sum(sq, axis=1, keepdims=True) * present
    denom = jnp.maximum(jnp.sum(present) * d, 1.0)
    out_ref[0, 0] = jnp.sum(per_class) / denom


def kernel(former_proto_list, former_proto_label, tf_feat_list, tf_label_list):
    n, d = tf_feat_list.shape
    c = former_proto_list.shape[0]
    nblk = n // CHUNK
    blk_per_tile = nblk // NW
    labels_i32 = tf_label_list.astype(jnp.int32)
    blk = jnp.arange(n, dtype=jnp.int32) // CHUNK
    wid = jnp.where(blk < blk_per_tile * NW, blk // blk_per_tile,
                    blk - blk_per_tile * NW)
    labels1d = labels_i32 + (wid % NS) * c
    zeros_acc = jnp.zeros((NS * c, d), jnp.float32)
    lrows = n // 128
    lrows_pad = ((lrows + HIST_BR - 1) // HIST_BR) * HIST_BR
    labels_pad = jnp.concatenate(
        [labels_i32, jnp.full((lrows_pad * 128 - n,), 127, jnp.int32)])
    psums = _sc_partial_sums(tf_feat_list, labels1d, zeros_acc, c)
    hist = _tc_histogram(labels_pad.reshape(lrows_pad, 128))
    psums = psums.reshape(NW, c, d)
    out = pl.pallas_call(
        _combine_kernel,
        out_shape=jax.ShapeDtypeStruct((1, 1), jnp.float32),
        out_specs=pl.BlockSpec(memory_space=pltpu.SMEM),
    )(psums, hist, former_proto_list, former_proto_label.astype(jnp.int32))
    return out[0, 0]


# speedup vs baseline: 1.0711x; 1.0711x over previous
import functools

import jax
import jax.numpy as jnp
from jax import lax
from jax.experimental import pallas as pl
from jax.experimental.pallas import tpu as pltpu
from jax.experimental.pallas import tpu_sc as plsc

NC = 2
NS = 16
NW = NC * NS
CHUNK = 80
NBUF = 10
PREFETCH = 5
HIST_BR = 256


def _sc_partial_sums(feats, labels1d, zeros_acc, c):
    n, d = feats.shape
    nblk = n // CHUNK
    blk_per_tile = nblk // NW
    extra = nblk - blk_per_tile * NW
    mesh = plsc.VectorSubcoreMesh(
        core_axis_name="c", subcore_axis_name="s", num_cores=NC, num_subcores=NS)

    @functools.partial(
        pl.kernel,
        out_type=jax.ShapeDtypeStruct((NC, NS * c, d), jnp.float32),
        mesh=mesh,
        scratch_types=[
            pltpu.VMEM((NBUF, CHUNK), jnp.int32),
            pltpu.VMEM((NBUF, CHUNK, d), jnp.float32),
            pltpu.VMEM_SHARED((NS * c, d), jnp.float32),
            pltpu.SemaphoreType.DMA((NBUF,)),
            pltpu.SemaphoreType.DMA((NBUF,)),
            pltpu.SemaphoreType.DMA((NBUF,)),
        ],
    )
    def run(feats_hbm, labels_hbm, zacc_hbm, out_sums, idxb, bufb,
            shared_acc, lsem, fsem, ssem):
        cid = lax.axis_index("c")
        sid = lax.axis_index("s")
        wid = cid * NS + sid
        @pl.when(sid == 0)
        def _():
            pltpu.sync_copy(zacc_hbm, shared_acc)

        plsc.subcore_barrier()

        def start(j):
            p = lax.rem(j, NBUF)
            blk = wid * blk_per_tile + j
            pltpu.async_copy(
                labels_hbm.at[pl.ds(blk * CHUNK, CHUNK)], idxb.at[p],
                lsem.at[p])
            pltpu.async_copy(
                feats_hbm.at[pl.ds(blk * CHUNK, CHUNK)], bufb.at[p],
                fsem.at[p])

        def wait_scatter(p):
            pltpu.make_async_copy(
                bufb.at[p], shared_acc.at[idxb.at[p]], ssem.at[p]).wait()

        for j0 in range(PREFETCH):
            start(j0)

        def body(j, carry):
            p = lax.rem(j, NBUF)
            pltpu.make_async_copy(
                labels_hbm.at[pl.ds(0, CHUNK)], idxb.at[p], lsem.at[p]).wait()
            pltpu.make_async_copy(
                feats_hbm.at[pl.ds(0, CHUNK)], bufb.at[p], fsem.at[p]).wait()
            pltpu.async_copy(bufb.at[p], shared_acc.at[idxb.at[p]], ssem.at[p],
                             add=True)
            m = j + PREFETCH

            @pl.when(m < blk_per_tile)
            def _():
                q = lax.rem(m, NBUF)

                @pl.when(m >= NBUF)
                def _():
                    wait_scatter(q)

                start(m)

            return carry

        lax.fori_loop(0, blk_per_tile, body, 0)
        for t in range(NBUF):
            wait_scatter((blk_per_tile - NBUF + t) % NBUF)

        @pl.when(wid < extra)
        def _():
            blk = blk_per_tile * NW + wid
            pltpu.sync_copy(labels_hbm.at[pl.ds(blk * CHUNK, CHUNK)],
                            idxb.at[0])
            pltpu.sync_copy(feats_hbm.at[pl.ds(blk * CHUNK, CHUNK)],
                            bufb.at[0])
            pltpu.sync_copy(bufb.at[0], shared_acc.at[idxb.at[0]], add=True)

        plsc.subcore_barrier()

        @pl.when(sid == 0)
        def _():
            pltpu.sync_copy(shared_acc, out_sums.at[cid])

    return run(feats, labels1d, zeros_acc)


def _hist_kernel(lab_ref, out_ref):
    i = pl.program_id(0)
    lab = lab_ref[...]
    class_row = lax.broadcasted_iota(jnp.int32, (HIST_BR, 128), 1)
    acc = jnp.zeros((1, 128), jnp.float32)
    for k in range(128):
        col = jnp.broadcast_to(lab[:, k:k + 1], (HIST_BR, 128))
        oh = (col == class_row).astype(jnp.float32)
        acc = acc + jnp.sum(oh, axis=0, keepdims=True)

    @pl.when(i == 0)
    def _():
        out_ref[...] = jnp.zeros_like(out_ref)

    out_ref[...] += jnp.broadcast_to(acc, out_ref.shape)


def _tc_histogram(labels_pad2d):
    rows = labels_pad2d.shape[0]
    return pl.pallas_call(
        _hist_kernel,
        grid=(rows // HIST_BR,),
        in_specs=[pl.BlockSpec((HIST_BR, 128), lambda i: (i, 0))],
        out_specs=pl.BlockSpec((8, 128), lambda i: (0, 0)),
        out_shape=jax.ShapeDtypeStruct((8, 128), jnp.float32),
    )(labels_pad2d)


def _combine_kernel(psums_ref, hist_ref, proto_ref, plabel_ref, out_ref):
    sums = jnp.sum(psums_ref[...], axis=0)
    c = sums.shape[0]
    d = sums.shape[1]
    labels = plabel_ref[...]
    onehot_c = (labels[:, None] ==
                lax.broadcasted_iota(jnp.int32, (c, c), 1)).astype(jnp.float32)
    onehot_k = (labels[:, None] ==
                lax.broadcasted_iota(jnp.int32, (c, 128), 1)).astype(jnp.float32)
    hist_sq = jnp.broadcast_to(hist_ref[0:1, :], (128, 128))
    eye = (lax.broadcasted_iota(jnp.int32, (128, 128), 0) ==
           lax.broadcasted_iota(jnp.int32, (128, 128), 1))
    hist_col = jnp.sum(jnp.where(eye, hist_sq, 0.0), axis=1,
                       keepdims=True)
    sums_sel = jnp.dot(onehot_c, sums,
                       preferred_element_type=jnp.float32,
                       precision=lax.Precision.HIGHEST)
    cnts_sel = jnp.dot(onehot_k, hist_col,
                       preferred_element_type=jnp.float32,
                       precision=lax.Precision.HIGHEST)
    safe = jnp.maximum(cnts_sel, 1.0)
    proto_cur = sums_sel / safe
    present = (cnts_sel > 0).astype(jnp.float32)
    sq = (proto_ref[...] - proto_cur) ** 2
    per_class = jnp.sum(sq, axis=1, keepdims=True) * present
    denom = jnp.maximum(jnp.sum(present) * d, 1.0)
    out_ref[0, 0] = jnp.sum(per_class) / denom


def kernel(former_proto_list, former_proto_label, tf_feat_list, tf_label_list):
    n, d = tf_feat_list.shape
    c = former_proto_list.shape[0]
    nblk = n // CHUNK
    blk_per_tile = nblk // NW
    labels_i32 = tf_label_list.astype(jnp.int32)
    blk = jnp.arange(n, dtype=jnp.int32) // CHUNK
    wid = jnp.where(blk < blk_per_tile * NW, blk // blk_per_tile,
                    blk - blk_per_tile * NW)
    labels1d = labels_i32 + (wid % NS) * c
    zeros_acc = jnp.zeros((NS * c, d), jnp.float32)
    lrows = n // 128
    lrows_pad = ((lrows + HIST_BR - 1) // HIST_BR) * HIST_BR
    labels_pad = jnp.concatenate(
        [labels_i32, jnp.full((lrows_pad * 128 - n,), 127, jnp.int32)])
    psums = _sc_partial_sums(tf_feat_list, labels1d, zeros_acc, c)
    hist = _tc_histogram(labels_pad.reshape(lrows_pad, 128))
    psums = psums.reshape(NW, c, d)
    out = pl.pallas_call(
        _combine_kernel,
        out_shape=jax.ShapeDtypeStruct((1, 1), jnp.float32),
        out_specs=pl.BlockSpec(memory_space=pltpu.SMEM),
    )(psums, hist, former_proto_list, former_proto_label.astype(jnp.int32))
    return out[0, 0]
